# baseline (speedup 1.0000x reference)
import functools
import math

import jax
import jax.numpy as jnp
from jax import lax
from jax.experimental import pallas as pl
from jax.experimental.pallas import tpu as pltpu
from jax.experimental.pallas import tpu_sc as plsc

_F = 1.0
_CH = 8


def kernel(x, exe_ids, pe):
    S, B, D = x.shape
    pe2 = jax.lax.slice(pe, (0, 0, 0), (S, 1, D)).reshape(S, D)
    ids_flat = exe_ids.reshape(S * B)
    NC, NS = 2, 16
    NW = NC * NS
    s_per_w = S // NW
    n_chunks = s_per_w // _CH
    mesh = plsc.VectorSubcoreMesh(core_axis_name="c", subcore_axis_name="s")

    @functools.partial(
        pl.kernel,
        mesh=mesh,
        out_type=jax.ShapeDtypeStruct((S, B, D), jnp.float32),
        scratch_types=[
            pltpu.VMEM((2, _CH, B, D), jnp.float32),
            pltpu.VMEM((2, _CH, D), jnp.float32),
            pltpu.VMEM((2, _CH * B), jnp.int32),
            pltpu.SemaphoreType.DMA,
            pltpu.SemaphoreType.DMA,
            pltpu.SemaphoreType.DMA,
            pltpu.SemaphoreType.DMA,
        ],
    )
    def k(x_hbm, ids_hbm, pe_hbm, out_hbm, xbuf, pebuf, idsbuf, sx, spe, sids, sout):
        wid = lax.axis_index("s") * NC + lax.axis_index("c")
        s_base = wid * s_per_w

        def start_in(c, slot):
            s0 = s_base + c * _CH
            pltpu.make_async_copy(
                x_hbm.at[pl.ds(s0, _CH)], xbuf.at[slot], sx).start()
            pltpu.make_async_copy(
                pe_hbm.at[pl.ds(s0, _CH)], pebuf.at[slot], spe).start()
            pltpu.make_async_copy(
                ids_hbm.at[pl.ds(s0 * B, _CH * B)], idsbuf.at[slot], sids).start()

        def wait_in():
            pltpu.make_async_copy(
                x_hbm.at[pl.ds(0, _CH)], xbuf.at[0], sx).wait()
            pltpu.make_async_copy(
                pe_hbm.at[pl.ds(0, _CH)], pebuf.at[0], spe).wait()
            pltpu.make_async_copy(
                ids_hbm.at[pl.ds(0, _CH * B)], idsbuf.at[0], sids).wait()

        start_in(0, 0)

        def chunk_body(c, carry):
            slot = lax.rem(c, 2)
            wait_in()

            @pl.when(c + 1 < n_chunks)
            def _():
                start_in(c + 1, 1 - slot)

            for q in range((_CH * B) // 16):
                idvec = idsbuf[slot, pl.ds(q * 16, 16)]
                scvec = jnp.where(
                    idvec != 0,
                    jnp.full((16,), 1.0 + _F, jnp.float32),
                    jnp.full((16,), 1.0, jnp.float32),
                )
                for ln in range(16):
                    r = q * 16 + ln
                    si, bi = r // B, r % B
                    scale_v = jax.lax.broadcast_in_dim(scvec[ln], (16,), ())

                    def lane_body(j, carry3, si=si, bi=bi, scale_v=scale_v):
                        xv = xbuf[slot, si, bi, pl.ds(j * 16, 16)]
                        pv = pebuf[slot, si, pl.ds(j * 16, 16)]
                        xbuf[slot, si, bi, pl.ds(j * 16, 16)] = xv + pv * scale_v
                        return carry3

                    lax.fori_loop(0, D // 16, lane_body, 0, unroll=8)

            s0 = s_base + c * _CH
            cp = pltpu.make_async_copy(
                xbuf.at[slot], out_hbm.at[pl.ds(s0, _CH)], sout)
            cp.start()
            cp.wait()
            return carry

        lax.fori_loop(0, n_chunks, chunk_body, 0)

    return k(x, ids_flat, pe2)


# speedup vs baseline: 4.7427x; 4.7427x over previous
import math

import jax
import jax.numpy as jnp
from jax.experimental import pallas as pl
from jax.experimental.pallas import tpu as pltpu

_EMPHASIS_FACTOR = 1.0
_HALF_PI = 0.5 * math.pi


def _body(x_ref, ids_ref, o_ref, sa_ref, ca_ref, sb_ref, cb_ref):
    i = pl.program_id(0)
    D = x_ref.shape[2]

    @pl.when(i == 0)
    def _build_tables():
        d = jax.lax.broadcasted_iota(jnp.int32, (1, D), 1)
        pair = (d >> 1) * 2
        w = jnp.exp(pair.astype(jnp.float32) * (-math.log(10000.0) / D))
        ph = (d & 1).astype(jnp.float32) * _HALF_PI
        m = jax.lax.broadcasted_iota(jnp.int32, (128, 1), 0).astype(jnp.float32)
        a_ang = (16.0 * m) * w + ph
        sa_ref[...] = jnp.sin(a_ang)
        ca_ref[...] = jnp.sin(a_ang + _HALF_PI)
        b = jax.lax.broadcasted_iota(jnp.int32, (16, 1), 0).astype(jnp.float32)
        b_ang = b * w
        sb_ref[...] = jnp.sin(b_ang)
        cb_ref[...] = jnp.sin(b_ang + _HALF_PI)

    sa = sa_ref[pl.ds(32 * i, 32), :][:, None, :]
    ca = ca_ref[pl.ds(32 * i, 32), :][:, None, :]
    sb = sb_ref[...][None, :, :]
    cb = cb_ref[...][None, :, :]
    pe = (sa * cb + ca * sb).reshape(x_ref.shape[0], x_ref.shape[2])
    scale = 1.0 + _EMPHASIS_FACTOR * (ids_ref[...] != 0).astype(jnp.float32)
    o_ref[...] = x_ref[...] + pe[:, None, :] * scale[:, :, None]


def kernel(x, exe_ids, pe):
    S, B, D = x.shape
    Sblk = 512
    grid = (S // Sblk,)
    return pl.pallas_call(
        _body,
        grid=grid,
        in_specs=[
            pl.BlockSpec((Sblk, B, D), lambda i: (i, 0, 0)),
            pl.BlockSpec((Sblk, B), lambda i: (i, 0)),
        ],
        out_specs=pl.BlockSpec((Sblk, B, D), lambda i: (i, 0, 0)),
        out_shape=jax.ShapeDtypeStruct((S, B, D), x.dtype),
        scratch_shapes=[
            pltpu.VMEM((S // 16, D), jnp.float32),
            pltpu.VMEM((S // 16, D), jnp.float32),
            pltpu.VMEM((16, D), jnp.float32),
            pltpu.VMEM((16, D), jnp.float32),
        ],
    )(x, exe_ids)
